# baseline (speedup 1.0000x reference)
import functools

import jax
import jax.numpy as jnp
from jax import lax
from jax.experimental import pallas as pl
from jax.experimental.pallas import tpu as pltpu
from jax.experimental.pallas import tpu_sc as plsc

D = 64
L = 16
NC, NS = 2, 16
NW = NC * NS
BBLK = 128
NG = BBLK // L

_NT, _NA, _NP, _NV, _NF = 20, 4, 8, 8, 3
_COMBO = _NT * _NA * _NP * _NV * _NF


def _combo_body(turn_ref, action_ref, pos_ref, civ_ref, face_ref, coinb_ref,
                out_ref, *, pos_off):
    def inner_rep(tbl, rep):
        n = tbl.shape[0]
        return jnp.broadcast_to(tbl[:, None, :], (n, rep, D)).reshape(n * rep, D)

    def outer_tile(tbl, times):
        r = tbl.shape[0]
        return jnp.broadcast_to(tbl[None], (times, r, D)).reshape(times * r, D)

    t = inner_rep(turn_ref[:], _NA * _NP * _NV * _NF)
    a = outer_tile(inner_rep(action_ref[:], _NP * _NV * _NF), _NT)
    p = outer_tile(inner_rep(pos_ref[pos_off:pos_off + _NP, :], _NV * _NF),
                   _NT * _NA)
    v = outer_tile(inner_rep(civ_ref[:], _NF), _NT * _NA * _NP)
    f = outer_tile(face_ref[:], _NT * _NA * _NP * _NV)
    out_ref[:] = t + a + p + v + f + coinb_ref[:]


def _build_combo(turn_table, action_table, pos_table, civ_table, face_table,
                 coin_b, pos_off):
    return pl.pallas_call(
        functools.partial(_combo_body, pos_off=pos_off),
        out_shape=jax.ShapeDtypeStruct((_COMBO, D), jnp.float32),
    )(turn_table, action_table, pos_table, civ_table, face_table,
      coin_b.reshape(1, D))


def _sc_body(x_hbm, combo_hbm, card_hbm, w_hbm, out_hbm, *s, ns, nb):
    (xb, ci, ti, cb, tb, ob, cn, wbuf) = (
        s[0:2], s[2:4], s[4:6], s[6:8], s[8:10], s[10:12], s[12:14], s[14])
    semx, semc, semt, semo = s[15:17], s[17:19], s[19:21], s[21:23]

    nchunks = ns
    wid = lax.axis_index("s") * NC + lax.axis_index("c")
    b0 = wid * BBLK

    pltpu.sync_copy(w_hbm, wbuf)
    wvec = [[wbuf[pl.ds(c * D + q * L, L)] for q in range(D // L)]
            for c in range(4)]

    iota16 = jnp.arange(L, dtype=jnp.int32)
    pat_minor = (iota16 % 8) * 128
    pat_major0 = iota16 // 8

    def bcast_lane(vec, k):
        idx = jnp.full((L, 1), k, jnp.int32)
        dnums = lax.GatherDimensionNumbers(
            offset_dims=(), collapsed_slice_dims=(0,), start_index_map=(0,))
        return lax.gather(vec, idx, dnums, (1,),
                          mode=lax.GatherScatterMode.PROMISE_IN_BOUNDS)

    def fire_x(c, p):
        pltpu.async_copy(x_hbm.at[:, pl.ds(c * nb + b0, BBLK)],
                         xb[p], semx[p])

    def wait_x(p):
        pltpu.make_async_copy(x_hbm.at[:, pl.ds(0, BBLK)], xb[p],
                              semx[p]).wait()

    def extract(p):
        for g in range(NG):
            sl = pl.ds(g * L, L)

            def col(j):
                return xb[p][j, sl]

            turn = col(0).astype(jnp.int32)
            card = col(1).astype(jnp.int32)
            act = col(2).astype(jnp.int32)
            pos = col(3).astype(jnp.int32)
            civ = col(4).astype(jnp.int32)
            face = col(5).astype(jnp.int32)
            combo = (((turn * _NA + act) * _NP + pos) * _NV + civ) * _NF + face
            ci[p][sl] = card
            ti[p][sl] = combo
            for cc in range(4):
                cn[p][cc, sl] = col(6 + cc)

    def fire_g(p):
        pltpu.async_copy(card_hbm.at[ci[p]], cb[p], semc[p])
        pltpu.async_copy(combo_hbm.at[ti[p]], tb[p], semt[p])

    def wait_g(p):
        pltpu.make_async_copy(card_hbm.at[ci[p]], cb[p], semc[p]).wait()
        pltpu.make_async_copy(combo_hbm.at[ti[p]], tb[p], semt[p]).wait()

    def fire_out(c, p):
        for dt in range(D // 8):
            pltpu.async_copy(ob[p].at[dt, :],
                             out_hbm.at[c * 256 + dt * 32 + wid, :], semo[p])

    def wait_out(p):
        for dt in range(D // 8):
            pltpu.make_async_copy(ob[p].at[dt, :], out_hbm.at[0, :],
                                  semo[p]).wait()

    def compute(p):
        def grp(g16, carry):
            coinv = [cn[p][cc, pl.ds(g16 * L, L)] for cc in range(4)]
            for k in range(L):
                t = g16 * L + k
                acc0 = cb[p][t, pl.ds(0, L)] + tb[p][t, pl.ds(0, L)]
                acc1 = cb[p][t, pl.ds(L, L)] + tb[p][t, pl.ds(L, L)]
                acc2 = cb[p][t, pl.ds(2 * L, L)] + tb[p][t, pl.ds(2 * L, L)]
                acc3 = cb[p][t, pl.ds(3 * L, L)] + tb[p][t, pl.ds(3 * L, L)]
                for cc in range(4):
                    cv = bcast_lane(coinv[cc], k)
                    acc0 = acc0 + cv * wvec[cc][0]
                    acc1 = acc1 + cv * wvec[cc][1]
                    acc2 = acc2 + cv * wvec[cc][2]
                    acc3 = acc3 + cv * wvec[cc][3]
                minor = pat_minor + t
                plsc.store_scatter(ob[p], [pat_major0, minor], acc0)
                plsc.store_scatter(ob[p], [pat_major0 + 2, minor], acc1)
                plsc.store_scatter(ob[p], [pat_major0 + 4, minor], acc2)
                plsc.store_scatter(ob[p], [pat_major0 + 6, minor], acc3)
            return carry

        lax.fori_loop(0, NG, grp, 0)

    fire_x(0, 0)
    wait_x(0)
    extract(0)
    fire_g(0)
    fire_x(1, 1)

    def pair(j, carry):
        for sgn in (0, 1):
            c = 2 * j + sgn
            p = sgn
            q = 1 - p

            @pl.when(c + 1 < nchunks)
            def _():
                wait_x(q)
                extract(q)
                fire_g(q)

            @pl.when(c + 2 < nchunks)
            def _():
                fire_x(c + 2, p)

            @pl.when(c < nchunks)
            def _():
                wait_g(p)

            @pl.when(jnp.logical_and(c >= 2, c < nchunks))
            def _():
                wait_out(p)

            @pl.when(c < nchunks)
            def _():
                compute(p)
                fire_out(c, p)

        return carry

    lax.fori_loop(0, (nchunks + 1) // 2, pair, 0)
    wait_out(1)
    wait_out(0)


def _sc_embed(x_t, combo, card_table, coin_w1d, n_s, n_b):
    mesh = plsc.VectorSubcoreMesh(core_axis_name="c", subcore_axis_name="s")
    dbl = lambda t: [t, t]
    return pl.kernel(
        functools.partial(_sc_body, ns=n_s, nb=n_b),
        out_type=jax.ShapeDtypeStruct((n_s * (D // 8) * (n_b // BBLK), 1024),
                                      jnp.float32),
        mesh=mesh,
        compiler_params=pltpu.CompilerParams(needs_layout_passes=False,
                                             use_tc_tiling_on_sc=False),
        scratch_types=(
            dbl(pltpu.VMEM((10, BBLK), jnp.float32))
            + dbl(pltpu.VMEM((BBLK,), jnp.int32))
            + dbl(pltpu.VMEM((BBLK,), jnp.int32))
            + dbl(pltpu.VMEM((BBLK, D), jnp.float32))
            + dbl(pltpu.VMEM((BBLK, D), jnp.float32))
            + dbl(pltpu.VMEM((D // 8, 8 * BBLK), jnp.float32))
            + dbl(pltpu.VMEM((4, BBLK), jnp.float32))
            + [pltpu.VMEM((4 * D,), jnp.float32)]
            + [pltpu.SemaphoreType.DMA] * 8
        ),
    )(x_t, combo, card_table, coin_w1d)


def kernel(x, turn_table, pos_table, civ_table, face_table, card_table,
           action_table, coin_W, coin_b):
    b, s, feat = x.shape
    assert feat == 10
    n = (s - 6) // 19
    pos_off = {3: 0, 4: 4, 5: 9, 6: 15, 7: 22}[int(n)]
    assert b % (NW * BBLK) == 0 or b == NW * BBLK

    combo = _build_combo(turn_table, action_table, pos_table, civ_table,
                         face_table, coin_b, pos_off)
    x_t = jnp.transpose(x, (2, 1, 0)).reshape(feat, s * b)
    res = _sc_embed(x_t, combo, card_table, coin_W.reshape(4 * D), s, b)
    out5 = res.reshape(s, D // 8, b // 128, 8, 128)
    return jnp.transpose(out5, (2, 4, 0, 1, 3)).reshape(b, s, D)


# speedup vs baseline: 1.1590x; 1.1590x over previous
import functools

import jax
import jax.numpy as jnp
from jax import lax
from jax.experimental import pallas as pl
from jax.experimental.pallas import tpu as pltpu
from jax.experimental.pallas import tpu_sc as plsc

D = 64
L = 16
NC, NS = 2, 16
NW = NC * NS
BBLK = 128
NG = BBLK // L

_NT, _NA, _NP, _NV, _NF = 20, 4, 8, 8, 3
_COMBO = _NT * _NA * _NP * _NV * _NF


def _combo_body(turn_ref, action_ref, pos_ref, civ_ref, face_ref, coinb_ref,
                out_ref, *, pos_off):
    def inner_rep(tbl, rep):
        n = tbl.shape[0]
        return jnp.broadcast_to(tbl[:, None, :], (n, rep, D)).reshape(n * rep, D)

    def outer_tile(tbl, times):
        r = tbl.shape[0]
        return jnp.broadcast_to(tbl[None], (times, r, D)).reshape(times * r, D)

    t = inner_rep(turn_ref[:], _NA * _NP * _NV * _NF)
    a = outer_tile(inner_rep(action_ref[:], _NP * _NV * _NF), _NT)
    p = outer_tile(inner_rep(pos_ref[pos_off:pos_off + _NP, :], _NV * _NF),
                   _NT * _NA)
    v = outer_tile(inner_rep(civ_ref[:], _NF), _NT * _NA * _NP)
    f = outer_tile(face_ref[:], _NT * _NA * _NP * _NV)
    out_ref[:] = t + a + p + v + f + coinb_ref[:]


def _build_combo(turn_table, action_table, pos_table, civ_table, face_table,
                 coin_b, pos_off):
    return pl.pallas_call(
        functools.partial(_combo_body, pos_off=pos_off),
        out_shape=jax.ShapeDtypeStruct((_COMBO, D), jnp.float32),
    )(turn_table, action_table, pos_table, civ_table, face_table,
      coin_b.reshape(1, D))


def _sc_body(x_hbm, combo_hbm, card_hbm, w_hbm, out_hbm, *s, ns, nb):
    (xb, ci, ti, cb, tb, ob, cn, wbuf, wsp) = (
        s[0:2], s[2:4], s[4:6], s[6:8], s[8:10], s[10:12], s[12:14], s[14],
        s[15])
    semx, semc, semt, semo = s[16:18], s[18:20], s[20:22], s[22:24]

    nchunks = ns
    wid = lax.axis_index("s") * NC + lax.axis_index("c")
    b0 = wid * BBLK

    iota16 = jnp.arange(L, dtype=jnp.int32)

    pltpu.sync_copy(w_hbm, wbuf)
    for cc in range(4):
        for d0 in range(D):
            wsp[cc, d0, :] = plsc.load_gather(
                wbuf, [cc * D + ((d0 + iota16) & (D - 1))])

    def fire_x(c, p):
        pltpu.async_copy(x_hbm.at[:, pl.ds(c * nb + b0, BBLK)],
                         xb[p], semx[p])

    def wait_x(p):
        pltpu.make_async_copy(x_hbm.at[:, pl.ds(0, BBLK)], xb[p],
                              semx[p]).wait()

    def extract(p):
        for g in range(NG):
            sl = pl.ds(g * L, L)

            def col(j):
                return xb[p][j, sl]

            turn = col(0).astype(jnp.int32)
            card = col(1).astype(jnp.int32)
            act = col(2).astype(jnp.int32)
            pos = col(3).astype(jnp.int32)
            civ = col(4).astype(jnp.int32)
            face = col(5).astype(jnp.int32)
            combo = (((turn * _NA + act) * _NP + pos) * _NV + civ) * _NF + face
            ci[p][sl] = card
            ti[p][sl] = combo
            for cc in range(4):
                cn[p][cc, sl] = col(6 + cc)

    def fire_g(p):
        pltpu.async_copy(card_hbm.at[ci[p]], cb[p], semc[p])
        pltpu.async_copy(combo_hbm.at[ti[p]], tb[p], semt[p])

    def wait_g(p):
        pltpu.make_async_copy(card_hbm.at[ci[p]], cb[p], semc[p]).wait()
        pltpu.make_async_copy(combo_hbm.at[ti[p]], tb[p], semt[p]).wait()

    def fire_out(c, p):
        for dt in range(D // 8):
            pltpu.async_copy(ob[p].at[pl.ds(dt * 1024, 1024)],
                             out_hbm.at[c * 256 + dt * 32 + wid, :], semo[p])

    def wait_out(p):
        for dt in range(D // 8):
            pltpu.make_async_copy(ob[p].at[pl.ds(0, 1024)], out_hbm.at[0, :],
                                  semo[p]).wait()

    def compute(p):
        coinv = [[cn[p][cc, pl.ds(g * L, L)] for g in range(NG)]
                 for cc in range(4)]
        tconst = [g * L + iota16 for g in range(NG)]

        def dbody(d0, carry):
            w0 = wsp[0, d0, :]
            w1 = wsp[1, d0, :]
            w2 = wsp[2, d0, :]
            w3 = wsp[3, d0, :]
            dmod = (d0 + iota16) & (D - 1)
            smbase = dmod * BBLK + iota16
            for g in range(NG):
                card = plsc.load_gather(cb[p], [tconst[g], dmod])
                comb = plsc.load_gather(tb[p], [tconst[g], dmod])
                acc = card + comb
                acc = acc + coinv[0][g] * w0
                acc = acc + coinv[1][g] * w1
                acc = acc + coinv[2][g] * w2
                acc = acc + coinv[3][g] * w3
                plsc.store_scatter(ob[p], [smbase + g * L], acc)
            return carry

        lax.fori_loop(0, D, dbody, 0)

    fire_x(0, 0)
    wait_x(0)
    extract(0)
    fire_g(0)
    fire_x(1, 1)

    def pair(j, carry):
        for sgn in (0, 1):
            c = 2 * j + sgn
            p = sgn
            q = 1 - p

            @pl.when(c + 1 < nchunks)
            def _():
                wait_x(q)
                extract(q)
                fire_g(q)

            @pl.when(c + 2 < nchunks)
            def _():
                fire_x(c + 2, p)

            @pl.when(c < nchunks)
            def _():
                wait_g(p)

            @pl.when(jnp.logical_and(c >= 2, c < nchunks))
            def _():
                wait_out(p)

            @pl.when(c < nchunks)
            def _():
                compute(p)
                fire_out(c, p)

        return carry

    lax.fori_loop(0, (nchunks + 1) // 2, pair, 0)
    wait_out(1)
    wait_out(0)


def _sc_embed(x_t, combo, card_table, coin_w1d, n_s, n_b):
    mesh = plsc.VectorSubcoreMesh(core_axis_name="c", subcore_axis_name="s")
    dbl = lambda t: [t, t]
    return pl.kernel(
        functools.partial(_sc_body, ns=n_s, nb=n_b),
        out_type=jax.ShapeDtypeStruct((n_s * (D // 8) * (n_b // BBLK), 1024),
                                      jnp.float32),
        mesh=mesh,
        compiler_params=pltpu.CompilerParams(needs_layout_passes=False,
                                             use_tc_tiling_on_sc=False),
        scratch_types=(
            dbl(pltpu.VMEM((10, BBLK), jnp.float32))
            + dbl(pltpu.VMEM((BBLK,), jnp.int32))
            + dbl(pltpu.VMEM((BBLK,), jnp.int32))
            + dbl(pltpu.VMEM((BBLK, D), jnp.float32))
            + dbl(pltpu.VMEM((BBLK, D), jnp.float32))
            + dbl(pltpu.VMEM((D * BBLK,), jnp.float32))
            + dbl(pltpu.VMEM((4, BBLK), jnp.float32))
            + [pltpu.VMEM((4 * D,), jnp.float32)]
            + [pltpu.VMEM((4, D, L), jnp.float32)]
            + [pltpu.SemaphoreType.DMA] * 8
        ),
    )(x_t, combo, card_table, coin_w1d)


def kernel(x, turn_table, pos_table, civ_table, face_table, card_table,
           action_table, coin_W, coin_b):
    b, s, feat = x.shape
    assert feat == 10
    n = (s - 6) // 19
    pos_off = {3: 0, 4: 4, 5: 9, 6: 15, 7: 22}[int(n)]
    assert b % (NW * BBLK) == 0 or b == NW * BBLK

    combo = _build_combo(turn_table, action_table, pos_table, civ_table,
                         face_table, coin_b, pos_off)
    x_t = jnp.transpose(x, (2, 1, 0)).reshape(feat, s * b)
    res = _sc_embed(x_t, combo, card_table, coin_W.reshape(4 * D), s, b)
    out5 = res.reshape(s, D // 8, b // 128, 8, 128)
    return jnp.transpose(out5, (2, 4, 0, 1, 3)).reshape(b, s, D)


# speedup vs baseline: 1.2106x; 1.0445x over previous
import functools

import jax
import jax.numpy as jnp
from jax import lax
from jax.experimental import pallas as pl
from jax.experimental.pallas import tpu as pltpu
from jax.experimental.pallas import tpu_sc as plsc

D = 64
L = 16
NC, NS = 2, 16
NW = NC * NS
BBLK = 128
NG = BBLK // L

_NT, _NA, _NP, _NV, _NF = 20, 4, 8, 8, 3
_COMBO = _NT * _NA * _NP * _NV * _NF


def _combo_body(turn_ref, action_ref, pos_ref, civ_ref, face_ref, coinb_ref,
                out_ref, *, pos_off):
    def inner_rep(tbl, rep):
        n = tbl.shape[0]
        return jnp.broadcast_to(tbl[:, None, :], (n, rep, D)).reshape(n * rep, D)

    def outer_tile(tbl, times):
        r = tbl.shape[0]
        return jnp.broadcast_to(tbl[None], (times, r, D)).reshape(times * r, D)

    t = inner_rep(turn_ref[:], _NA * _NP * _NV * _NF)
    a = outer_tile(inner_rep(action_ref[:], _NP * _NV * _NF), _NT)
    p = outer_tile(inner_rep(pos_ref[pos_off:pos_off + _NP, :], _NV * _NF),
                   _NT * _NA)
    v = outer_tile(inner_rep(civ_ref[:], _NF), _NT * _NA * _NP)
    f = outer_tile(face_ref[:], _NT * _NA * _NP * _NV)
    out_ref[:] = t + a + p + v + f + coinb_ref[:]


def _build_combo(turn_table, action_table, pos_table, civ_table, face_table,
                 coin_b, pos_off):
    return pl.pallas_call(
        functools.partial(_combo_body, pos_off=pos_off),
        out_shape=jax.ShapeDtypeStruct((_COMBO, D), jnp.float32),
    )(turn_table, action_table, pos_table, civ_table, face_table,
      coin_b.reshape(1, D))


def _sc_body(x_hbm, combo_hbm, card_hbm, w_hbm, out_hbm, *s, ns, nb):
    (xb, ci, ti, cb, tb, ob, cn, wbuf, wsp) = (
        s[0:2], s[2:4], s[4:6], s[6:8], s[8:10], s[10:12], s[12:14], s[14],
        s[15])
    semx, semc, semt, semo = s[16:18], s[18:20], s[20:22], s[22:24]

    nchunks = ns
    wid = lax.axis_index("s") * NC + lax.axis_index("c")
    b0 = wid * BBLK

    iota16 = jnp.arange(L, dtype=jnp.int32)

    pltpu.sync_copy(w_hbm, wbuf)
    for cc in range(4):
        for d0 in range(D):
            wsp[cc, d0, :] = plsc.load_gather(
                wbuf, [cc * D + ((d0 + iota16) & (D - 1))])

    def fire_x(c, p):
        pltpu.async_copy(x_hbm.at[:, pl.ds(c * nb + b0, BBLK)],
                         xb[p], semx[p])

    def wait_x(p):
        pltpu.make_async_copy(x_hbm.at[:, pl.ds(0, BBLK)], xb[p],
                              semx[p]).wait()

    def extract(p):
        for g in range(NG):
            sl = pl.ds(g * L, L)

            def col(j):
                return xb[p][j, sl]

            turn = col(0).astype(jnp.int32)
            card = col(1).astype(jnp.int32)
            act = col(2).astype(jnp.int32)
            pos = col(3).astype(jnp.int32)
            civ = col(4).astype(jnp.int32)
            face = col(5).astype(jnp.int32)
            combo = (((turn * _NA + act) * _NP + pos) * _NV + civ) * _NF + face
            ci[p][sl] = card
            ti[p][sl] = combo
            for cc in range(4):
                cn[p][cc, sl] = col(6 + cc)

    def fire_g(p):
        pltpu.async_copy(card_hbm.at[ci[p]], cb[p], semc[p])
        pltpu.async_copy(combo_hbm.at[ti[p]], tb[p], semt[p])

    def wait_g(p):
        pltpu.make_async_copy(card_hbm.at[ci[p]], cb[p], semc[p]).wait()
        pltpu.make_async_copy(combo_hbm.at[ti[p]], tb[p], semt[p]).wait()

    def fire_out(c, p):
        for dt in range(D // 8):
            pltpu.async_copy(ob[p].at[pl.ds(dt * 1024, 1024)],
                             out_hbm.at[c * 256 + dt * 32 + wid, :], semo[p])

    def wait_out(p):
        for dt in range(D // 8):
            pltpu.make_async_copy(ob[p].at[pl.ds(0, 1024)], out_hbm.at[0, :],
                                  semo[p]).wait()

    def compute(p):
        tconst = [g * L + iota16 for g in range(NG)]

        def dbody(d0, carry):
            w0 = wsp[0, d0, :]
            w1 = wsp[1, d0, :]
            w2 = wsp[2, d0, :]
            w3 = wsp[3, d0, :]
            dmod = (d0 + iota16) & (D - 1)
            smbase = dmod * BBLK + iota16
            for g in range(NG):
                sl = pl.ds(g * L, L)
                card = plsc.load_gather(cb[p], [tconst[g], dmod])
                comb = plsc.load_gather(tb[p], [tconst[g], dmod])
                m01 = cn[p][0, sl] * w0 + cn[p][1, sl] * w1
                m23 = cn[p][2, sl] * w2 + cn[p][3, sl] * w3
                acc = (card + comb) + (m01 + m23)
                plsc.store_scatter(ob[p], [smbase + g * L], acc)
            return carry

        lax.fori_loop(0, D, dbody, 0, unroll=2)

    fire_x(0, 0)
    wait_x(0)
    extract(0)
    fire_g(0)
    fire_x(1, 1)

    def pair(j, carry):
        for sgn in (0, 1):
            c = 2 * j + sgn
            p = sgn
            q = 1 - p

            @pl.when(c + 1 < nchunks)
            def _():
                wait_x(q)
                extract(q)
                fire_g(q)

            @pl.when(c + 2 < nchunks)
            def _():
                fire_x(c + 2, p)

            @pl.when(c < nchunks)
            def _():
                wait_g(p)

            @pl.when(jnp.logical_and(c >= 2, c < nchunks))
            def _():
                wait_out(p)

            @pl.when(c < nchunks)
            def _():
                compute(p)
                fire_out(c, p)

        return carry

    lax.fori_loop(0, (nchunks + 1) // 2, pair, 0)
    wait_out(1)
    wait_out(0)


def _sc_embed(x_t, combo, card_table, coin_w1d, n_s, n_b):
    mesh = plsc.VectorSubcoreMesh(core_axis_name="c", subcore_axis_name="s")
    dbl = lambda t: [t, t]
    return pl.kernel(
        functools.partial(_sc_body, ns=n_s, nb=n_b),
        out_type=jax.ShapeDtypeStruct((n_s * (D // 8) * (n_b // BBLK), 1024),
                                      jnp.float32),
        mesh=mesh,
        compiler_params=pltpu.CompilerParams(needs_layout_passes=False,
                                             use_tc_tiling_on_sc=False),
        scratch_types=(
            dbl(pltpu.VMEM((10, BBLK), jnp.float32))
            + dbl(pltpu.VMEM((BBLK,), jnp.int32))
            + dbl(pltpu.VMEM((BBLK,), jnp.int32))
            + dbl(pltpu.VMEM((BBLK, D), jnp.float32))
            + dbl(pltpu.VMEM((BBLK, D), jnp.float32))
            + dbl(pltpu.VMEM((D * BBLK,), jnp.float32))
            + dbl(pltpu.VMEM((4, BBLK), jnp.float32))
            + [pltpu.VMEM((4 * D,), jnp.float32)]
            + [pltpu.VMEM((4, D, L), jnp.float32)]
            + [pltpu.SemaphoreType.DMA] * 8
        ),
    )(x_t, combo, card_table, coin_w1d)


def kernel(x, turn_table, pos_table, civ_table, face_table, card_table,
           action_table, coin_W, coin_b):
    b, s, feat = x.shape
    assert feat == 10
    n = (s - 6) // 19
    pos_off = {3: 0, 4: 4, 5: 9, 6: 15, 7: 22}[int(n)]
    assert b % (NW * BBLK) == 0 or b == NW * BBLK

    combo = _build_combo(turn_table, action_table, pos_table, civ_table,
                         face_table, coin_b, pos_off)
    x_t = jnp.transpose(x, (2, 1, 0)).reshape(feat, s * b)
    res = _sc_embed(x_t, combo, card_table, coin_W.reshape(4 * D), s, b)
    out5 = res.reshape(s, D // 8, b // 128, 8, 128)
    return jnp.transpose(out5, (2, 4, 0, 1, 3)).reshape(b, s, D)


# speedup vs baseline: 2.2078x; 1.8238x over previous
import functools

import jax
import jax.numpy as jnp
from jax import lax
from jax.experimental import pallas as pl
from jax.experimental.pallas import tpu as pltpu
from jax.experimental.pallas import tpu_sc as plsc

D = 64
L = 16
NC, NS = 2, 16
NW = NC * NS
BBLK = 128
NG = BBLK // L

_NT, _NA, _NP, _NV, _NF = 20, 4, 8, 8, 3
_COMBO = _NT * _NA * _NP * _NV * _NF


def _combo_body(turn_ref, action_ref, pos_ref, civ_ref, face_ref, coinb_ref,
                out_ref, *, pos_off):
    def inner_rep(tbl, rep):
        n = tbl.shape[0]
        return jnp.broadcast_to(tbl[:, None, :], (n, rep, D)).reshape(n * rep, D)

    def outer_tile(tbl, times):
        r = tbl.shape[0]
        return jnp.broadcast_to(tbl[None], (times, r, D)).reshape(times * r, D)

    t = inner_rep(turn_ref[:], _NA * _NP * _NV * _NF)
    a = outer_tile(inner_rep(action_ref[:], _NP * _NV * _NF), _NT)
    p = outer_tile(inner_rep(pos_ref[pos_off:pos_off + _NP, :], _NV * _NF),
                   _NT * _NA)
    v = outer_tile(inner_rep(civ_ref[:], _NF), _NT * _NA * _NP)
    f = outer_tile(face_ref[:], _NT * _NA * _NP * _NV)
    out_ref[:] = t + a + p + v + f + coinb_ref[:]


def _build_combo(turn_table, action_table, pos_table, civ_table, face_table,
                 coin_b, pos_off):
    return pl.pallas_call(
        functools.partial(_combo_body, pos_off=pos_off),
        out_shape=jax.ShapeDtypeStruct((_COMBO, D), jnp.float32),
    )(turn_table, action_table, pos_table, civ_table, face_table,
      coin_b.reshape(1, D))


def _sc_body(x_hbm, combo_hbm, card_hbm, w_hbm, out_hbm, *s, ns, nb):
    (xb, ci, ti, cb, tb, ob, cn, wbuf, wsp) = (
        s[0:2], s[2:4], s[4:6], s[6:8], s[8:10], s[10:12], s[12:14], s[14],
        s[15])
    semx, semc, semt, semo = s[16:18], s[18:20], s[20:22], s[22:24]

    nchunks = ns
    wid = lax.axis_index("s") * NC + lax.axis_index("c")
    b0 = wid * BBLK

    iota16 = jnp.arange(L, dtype=jnp.int32)

    pltpu.sync_copy(w_hbm, wbuf)
    for cc in range(4):
        for d0 in range(D):
            wsp[cc, d0, :] = plsc.load_gather(
                wbuf, [cc * D + ((d0 + iota16) & (D - 1))])

    def fire_x(c, p):
        pltpu.async_copy(x_hbm.at[:, pl.ds(c * nb + b0, BBLK)],
                         xb[p], semx[p])

    def wait_x(p):
        pltpu.make_async_copy(x_hbm.at[:, pl.ds(0, BBLK)], xb[p],
                              semx[p]).wait()

    def extract(p):
        for g in range(NG):
            sl = pl.ds(g * L, L)

            def col(j):
                return xb[p][j, sl]

            turn = col(0).astype(jnp.int32)
            card = col(1).astype(jnp.int32)
            act = col(2).astype(jnp.int32)
            pos = col(3).astype(jnp.int32)
            civ = col(4).astype(jnp.int32)
            face = col(5).astype(jnp.int32)
            combo = (((turn * _NA + act) * _NP + pos) * _NV + civ) * _NF + face
            ci[p][sl] = card
            ti[p][sl] = combo
            for cc in range(4):
                cn[p][cc, sl] = col(6 + cc)

    def fire_g(p):
        pltpu.async_copy(card_hbm.at[ci[p]], cb[p], semc[p])
        pltpu.async_copy(combo_hbm.at[ti[p]], tb[p], semt[p])

    def wait_g(p):
        pltpu.make_async_copy(card_hbm.at[ci[p]], cb[p], semc[p]).wait()
        pltpu.make_async_copy(combo_hbm.at[ti[p]], tb[p], semt[p]).wait()

    def fire_out(c, p):
        for dt in range(D // 8):
            pltpu.async_copy(ob[p].at[pl.ds(dt * 1024, 1024)],
                             out_hbm.at[c * 256 + dt * 32 + wid, :], semo[p])

    def wait_out(p):
        for dt in range(D // 8):
            pltpu.make_async_copy(ob[p].at[pl.ds(0, 1024)], out_hbm.at[0, :],
                                  semo[p]).wait()

    def compute(p):
        tconst = [g * L + iota16 for g in range(NG)]

        def dbody(d0, carry):
            w0 = wsp[0, d0, :]
            w1 = wsp[1, d0, :]
            w2 = wsp[2, d0, :]
            w3 = wsp[3, d0, :]
            dmod = (d0 + iota16) & (D - 1)
            smbase = dmod * BBLK + iota16
            cards = [plsc.load_gather(cb[p], [tconst[g], dmod])
                     for g in range(NG)]
            combs = [plsc.load_gather(tb[p], [tconst[g], dmod])
                     for g in range(NG)]
            accs = []
            for g in range(NG):
                sl = pl.ds(g * L, L)
                m01 = cn[p][0, sl] * w0 + cn[p][1, sl] * w1
                m23 = cn[p][2, sl] * w2 + cn[p][3, sl] * w3
                accs.append((cards[g] + combs[g]) + (m01 + m23))
            for g in range(NG):
                plsc.store_scatter(ob[p], [smbase + g * L], accs[g])
            return carry

        lax.fori_loop(0, D, dbody, 0, unroll=2)

    fire_x(0, 0)
    wait_x(0)
    extract(0)
    fire_g(0)
    fire_x(1, 1)

    def pair(j, carry):
        for sgn in (0, 1):
            c = 2 * j + sgn
            p = sgn
            q = 1 - p

            @pl.when(c + 1 < nchunks)
            def _():
                wait_x(q)
                extract(q)
                fire_g(q)

            @pl.when(c + 2 < nchunks)
            def _():
                fire_x(c + 2, p)

            @pl.when(c < nchunks)
            def _():
                wait_g(p)

            @pl.when(jnp.logical_and(c >= 2, c < nchunks))
            def _():
                wait_out(p)

            @pl.when(c < nchunks)
            def _():
                compute(p)
                fire_out(c, p)

        return carry

    lax.fori_loop(0, (nchunks + 1) // 2, pair, 0)
    wait_out(1)
    wait_out(0)


def _sc_embed(x_t, combo, card_table, coin_w1d, n_s, n_b):
    mesh = plsc.VectorSubcoreMesh(core_axis_name="c", subcore_axis_name="s")
    dbl = lambda t: [t, t]
    return pl.kernel(
        functools.partial(_sc_body, ns=n_s, nb=n_b),
        out_type=jax.ShapeDtypeStruct((n_s * (D // 8) * (n_b // BBLK), 1024),
                                      jnp.float32),
        mesh=mesh,
        compiler_params=pltpu.CompilerParams(needs_layout_passes=False,
                                             use_tc_tiling_on_sc=False),
        scratch_types=(
            dbl(pltpu.VMEM((10, BBLK), jnp.float32))
            + dbl(pltpu.VMEM((BBLK,), jnp.int32))
            + dbl(pltpu.VMEM((BBLK,), jnp.int32))
            + dbl(pltpu.VMEM((BBLK, D), jnp.float32))
            + dbl(pltpu.VMEM((BBLK, D), jnp.float32))
            + dbl(pltpu.VMEM((D * BBLK,), jnp.float32))
            + dbl(pltpu.VMEM((4, BBLK), jnp.float32))
            + [pltpu.VMEM((4 * D,), jnp.float32)]
            + [pltpu.VMEM((4, D, L), jnp.float32)]
            + [pltpu.SemaphoreType.DMA] * 8
        ),
    )(x_t, combo, card_table, coin_w1d)


def kernel(x, turn_table, pos_table, civ_table, face_table, card_table,
           action_table, coin_W, coin_b):
    b, s, feat = x.shape
    assert feat == 10
    n = (s - 6) // 19
    pos_off = {3: 0, 4: 4, 5: 9, 6: 15, 7: 22}[int(n)]
    assert b % (NW * BBLK) == 0 or b == NW * BBLK

    combo = _build_combo(turn_table, action_table, pos_table, civ_table,
                         face_table, coin_b, pos_off)
    x_t = jnp.transpose(x, (2, 1, 0)).reshape(feat, s * b)
    res = _sc_embed(x_t, combo, card_table, coin_W.reshape(4 * D), s, b)
    out5 = res.reshape(s, D // 8, b // 128, 8, 128)
    return jnp.transpose(out5, (2, 4, 0, 1, 3)).reshape(b, s, D)
